# baseline (speedup 1.0000x reference)
import jax
import jax.numpy as jnp
from jax.experimental import pallas as pl
from jax.experimental.pallas import tpu as pltpu

B, ROI, T, H, F = 4, 90, 64, 3, 128
HID = 128
R = ROI * H
G = 3 * HID
BT = T * B
N_ROWS = BT * ROI
TC1 = 4
RT = 9
NRC = 3
RC = R // NRC
TC4 = 4


def _k1_body(a_ref, x_ref, w1_ref, b1_ref, eps_ref, z1_ref, st_ref, acc_ref):
    b = pl.program_id(0)
    tt = pl.program_id(1)

    @pl.when(jnp.logical_and(b == 0, tt == 0))
    def _():
        acc_ref[...] = jnp.zeros_like(acc_ref)

    for tj in range(TC1):
        for h in range(H):
            adj = (a_ref[0, :, tj, h, :] != 0.0).astype(jnp.float32)
            hx = x_ref[0, :, tj, h, :]
            agg = jnp.dot(adj, hx, preferred_element_type=jnp.float32) + eps_ref[h, 0, 0] * hx
            z = jnp.dot(agg, w1_ref[h], preferred_element_type=jnp.float32) + b1_ref[h]
            z1_ref[h, 0, tj] = z
            acc_ref[h, 0:1, :] += jnp.sum(z, axis=0, keepdims=True)
            acc_ref[h, 1:2, :] += jnp.sum(z * z, axis=0, keepdims=True)

    @pl.when(jnp.logical_and(b == B - 1, tt == T // TC1 - 1))
    def _():
        st_ref[...] = acc_ref[...]


def _k2_body(z1_ref, sc1_ref, sh1_ref, w2_ref, b2_ref, z2_ref, st_ref, acc_ref):
    i = pl.program_id(1)

    @pl.when(i == 0)
    def _():
        acc_ref[...] = jnp.zeros_like(acc_ref)

    u = z1_ref[0].reshape(RT * BT, F) * sc1_ref[0] + sh1_ref[0]
    u = jnp.where(u > 0.0, u, jnp.exp(u) - 1.0)
    z2 = jnp.dot(u, w2_ref[0], preferred_element_type=jnp.float32) + b2_ref[0]
    z2_ref[0] = z2.reshape(RT, BT, F)
    acc_ref[0:1, :] += jnp.sum(z2, axis=0, keepdims=True)
    acc_ref[1:2, :] += jnp.sum(z2 * z2, axis=0, keepdims=True)

    @pl.when(i == ROI // RT - 1)
    def _():
        st_ref[0] = acc_ref[...]


def _k3_body(z2_ref, sc2_ref, sh2_ref, wih_ref, bih_ref, gi_ref):
    for j in range(RT):
        h, roi = j % H, j // H
        u = jnp.maximum(z2_ref[h, roi] * sc2_ref[h] + sh2_ref[h], 0.0)
        gi = jax.lax.dot_general(
            u, wih_ref[j],
            dimension_numbers=(((1,), (1,)), ((), ())),
            preferred_element_type=jnp.float32,
        ) + bih_ref[j]
        gi_ref[:, j] = gi.reshape(T, B, G)


def _k4_body(gi_ref, whh_ref, bhh_ref, y_ref, h_ref):
    tt = pl.program_id(1)

    @pl.when(tt == 0)
    def _():
        h_ref[...] = jnp.zeros_like(h_ref)

    hp = h_ref[...]
    wb = whh_ref[...].astype(jnp.bfloat16)
    for tj in range(TC4):
        gh = jax.lax.dot_general(
            hp.astype(jnp.bfloat16), wb,
            dimension_numbers=(((2,), (1,)), ((0,), (0,))),
            preferred_element_type=jnp.float32,
        ) + bhh_ref[...]
        gi = gi_ref[tj]
        rg = jax.nn.sigmoid(gi[:, :, :HID] + gh[:, :, :HID])
        zg = jax.nn.sigmoid(gi[:, :, HID:2 * HID] + gh[:, :, HID:2 * HID])
        ng = jnp.tanh(gi[:, :, 2 * HID:] + rg * gh[:, :, 2 * HID:])
        hp = (1.0 - zg) * ng + zg * hp
        y_ref[tj] = hp
    h_ref[...] = hp


def _bn_affine(st, g, be):
    n = jnp.float32(N_ROWS)
    mean = st[:, 0, :] / n
    var = st[:, 1, :] / n - mean * mean
    rstd = jax.lax.rsqrt(var + 1e-5)
    scale = g * rstd
    shift = be - mean * scale
    return scale.reshape(H, 1, F), shift.reshape(H, 1, F)


def kernel(x, a, gnn_W1, gnn_b1, gnn_g1, gnn_be1, gnn_W2, gnn_b2, gnn_g2,
           gnn_be2, gnn_eps, gru_Wih, gru_Whh, gru_bih, gru_bhh):
    z1, st1 = pl.pallas_call(
        _k1_body,
        grid=(B, T // TC1),
        in_specs=[
            pl.BlockSpec((1, ROI, TC1, H, ROI), lambda b, t: (b, 0, t, 0, 0)),
            pl.BlockSpec((1, ROI, TC1, H, F), lambda b, t: (b, 0, t, 0, 0)),
            pl.BlockSpec((H, F, F), lambda b, t: (0, 0, 0)),
            pl.BlockSpec((H, 1, F), lambda b, t: (0, 0, 0)),
            pl.BlockSpec((H, 1, 1), lambda b, t: (0, 0, 0)),
        ],
        out_specs=[
            pl.BlockSpec((H, 1, TC1, ROI, F), lambda b, t: (0, b, t, 0, 0)),
            pl.BlockSpec((H, 8, F), lambda b, t: (0, 0, 0)),
        ],
        out_shape=[
            jax.ShapeDtypeStruct((H, B, T, ROI, F), jnp.float32),
            jax.ShapeDtypeStruct((H, 8, F), jnp.float32),
        ],
        scratch_shapes=[pltpu.VMEM((H, 8, F), jnp.float32)],
    )(a, x, gnn_W1, gnn_b1.reshape(H, 1, F), gnn_eps.reshape(H, 1, 1))

    scale1, shift1 = _bn_affine(st1, gnn_g1, gnn_be1)
    z1p = jnp.transpose(z1, (0, 3, 2, 1, 4)).reshape(H, ROI, BT, F)

    z2r, st2 = pl.pallas_call(
        _k2_body,
        grid=(H, ROI // RT),
        in_specs=[
            pl.BlockSpec((1, RT, BT, F), lambda h, r: (h, r, 0, 0)),
            pl.BlockSpec((1, 1, F), lambda h, r: (h, 0, 0)),
            pl.BlockSpec((1, 1, F), lambda h, r: (h, 0, 0)),
            pl.BlockSpec((1, F, F), lambda h, r: (h, 0, 0)),
            pl.BlockSpec((1, 1, F), lambda h, r: (h, 0, 0)),
        ],
        out_specs=[
            pl.BlockSpec((1, RT, BT, F), lambda h, r: (h, r, 0, 0)),
            pl.BlockSpec((1, 8, F), lambda h, r: (h, 0, 0)),
        ],
        out_shape=[
            jax.ShapeDtypeStruct((H, ROI, BT, F), jnp.float32),
            jax.ShapeDtypeStruct((H, 8, F), jnp.float32),
        ],
        scratch_shapes=[pltpu.VMEM((8, F), jnp.float32)],
    )(z1p, scale1, shift1, gnn_W2, gnn_b2.reshape(H, 1, F))

    scale2, shift2 = _bn_affine(st2, gnn_g2, gnn_be2)

    gi = pl.pallas_call(
        _k3_body,
        grid=(R // RT,),
        in_specs=[
            pl.BlockSpec((H, RT // H, BT, F), lambda c: (0, c, 0, 0)),
            pl.BlockSpec((H, 1, F), lambda c: (0, 0, 0)),
            pl.BlockSpec((H, 1, F), lambda c: (0, 0, 0)),
            pl.BlockSpec((RT, G, F), lambda c: (c, 0, 0)),
            pl.BlockSpec((RT, 1, G), lambda c: (c, 0, 0)),
        ],
        out_specs=pl.BlockSpec((T, RT, B, G), lambda c: (0, c, 0, 0)),
        out_shape=jax.ShapeDtypeStruct((T, R, B, G), jnp.float32),
    )(z2r, scale2, shift2, gru_Wih, gru_bih.reshape(R, 1, G))

    gif = gi
    whh_h = jnp.transpose(gru_Whh, (0, 2, 1))
    bhh_h = gru_bhh.reshape(R, 1, G)

    y = pl.pallas_call(
        _k4_body,
        grid=(NRC, T // TC4),
        in_specs=[
            pl.BlockSpec((TC4, RC, B, G), lambda c, t: (t, c, 0, 0)),
            pl.BlockSpec((RC, HID, G), lambda c, t: (c, 0, 0)),
            pl.BlockSpec((RC, 1, G), lambda c, t: (c, 0, 0)),
        ],
        out_specs=pl.BlockSpec((TC4, RC, B, HID), lambda c, t: (t, c, 0, 0)),
        out_shape=jax.ShapeDtypeStruct((T, R, B, HID), jnp.float32),
        scratch_shapes=[pltpu.VMEM((RC, B, HID), jnp.float32)],
    )(gif, whh_h, bhh_h)

    return jnp.transpose(y.reshape(T, ROI, H, B, HID), (3, 1, 0, 2, 4))


# speedup vs baseline: 1.0611x; 1.0611x over previous
import jax
import jax.numpy as jnp
from jax.experimental import pallas as pl
from jax.experimental.pallas import tpu as pltpu

B, ROI, T, H, F = 4, 90, 64, 3, 128
HID = 128
R = ROI * H
G = 3 * HID
BT = T * B
N_ROWS = BT * ROI
TC1 = 4
RT = 9
NRC = 3
RC = R // NRC
TC4 = 4


def _k1_body(a_ref, x_ref, w1_ref, b1_ref, eps_ref, z1_ref, st_ref, acc_ref):
    b = pl.program_id(0)
    tt = pl.program_id(1)

    @pl.when(jnp.logical_and(b == 0, tt == 0))
    def _():
        acc_ref[...] = jnp.zeros_like(acc_ref)

    for tj in range(TC1):
        for h in range(H):
            adj = (a_ref[0, :, tj, h, :] != 0.0).astype(jnp.float32)
            hx = x_ref[0, :, tj, h, :]
            agg = jnp.dot(adj, hx, preferred_element_type=jnp.float32) + eps_ref[h, 0, 0] * hx
            z = jnp.dot(agg, w1_ref[h], preferred_element_type=jnp.float32) + b1_ref[h]
            z1_ref[h, 0, tj] = z.astype(jnp.bfloat16)
            acc_ref[h, 0:1, :] += jnp.sum(z, axis=0, keepdims=True)
            acc_ref[h, 1:2, :] += jnp.sum(z * z, axis=0, keepdims=True)

    @pl.when(jnp.logical_and(b == B - 1, tt == T // TC1 - 1))
    def _():
        st_ref[...] = acc_ref[...]


def _k2_body(z1_ref, sc1_ref, sh1_ref, w2_ref, b2_ref, z2_ref, st_ref, acc_ref):
    i = pl.program_id(1)

    @pl.when(i == 0)
    def _():
        acc_ref[...] = jnp.zeros_like(acc_ref)

    u = z1_ref[0].reshape(RT * BT, F).astype(jnp.float32) * sc1_ref[0] + sh1_ref[0]
    u = jnp.where(u > 0.0, u, jnp.exp(u) - 1.0)
    z2 = jnp.dot(u, w2_ref[0], preferred_element_type=jnp.float32) + b2_ref[0]
    z2_ref[0] = z2.astype(jnp.bfloat16).reshape(RT, BT, F)
    acc_ref[0:1, :] += jnp.sum(z2, axis=0, keepdims=True)
    acc_ref[1:2, :] += jnp.sum(z2 * z2, axis=0, keepdims=True)

    @pl.when(i == ROI // RT - 1)
    def _():
        st_ref[0] = acc_ref[...]


def _k3_body(z2_ref, sc2_ref, sh2_ref, wih_ref, bih_ref, gi_ref):
    for j in range(RT):
        h, roi = j % H, j // H
        u = jnp.maximum(z2_ref[h, roi].astype(jnp.float32) * sc2_ref[h] + sh2_ref[h], 0.0)
        gi = jax.lax.dot_general(
            u, wih_ref[j],
            dimension_numbers=(((1,), (1,)), ((), ())),
            preferred_element_type=jnp.float32,
        ) + bih_ref[j]
        gi_ref[:, j] = gi.astype(jnp.bfloat16).reshape(T, B, G)


def _k4_body(gi_ref, whh_ref, bhh_ref, y_ref, h_ref):
    tt = pl.program_id(1)

    @pl.when(tt == 0)
    def _():
        h_ref[...] = jnp.zeros_like(h_ref)

    hp = h_ref[...]
    for tj in range(TC4):
        gh = jax.lax.dot_general(
            hp.astype(jnp.bfloat16), whh_ref[...],
            dimension_numbers=(((2,), (1,)), ((0,), (0,))),
            preferred_element_type=jnp.float32,
        ) + bhh_ref[...]
        gi = gi_ref[tj].astype(jnp.float32)
        rg = jax.nn.sigmoid(gi[:, :, :HID] + gh[:, :, :HID])
        zg = jax.nn.sigmoid(gi[:, :, HID:2 * HID] + gh[:, :, HID:2 * HID])
        ng = jnp.tanh(gi[:, :, 2 * HID:] + rg * gh[:, :, 2 * HID:])
        hp = (1.0 - zg) * ng + zg * hp
        y_ref[tj] = hp
    h_ref[...] = hp


def _bn_affine(st, g, be):
    n = jnp.float32(N_ROWS)
    mean = st[:, 0, :] / n
    var = st[:, 1, :] / n - mean * mean
    rstd = jax.lax.rsqrt(var + 1e-5)
    scale = g * rstd
    shift = be - mean * scale
    return scale.reshape(H, 1, F), shift.reshape(H, 1, F)


def kernel(x, a, gnn_W1, gnn_b1, gnn_g1, gnn_be1, gnn_W2, gnn_b2, gnn_g2,
           gnn_be2, gnn_eps, gru_Wih, gru_Whh, gru_bih, gru_bhh):
    z1, st1 = pl.pallas_call(
        _k1_body,
        grid=(B, T // TC1),
        in_specs=[
            pl.BlockSpec((1, ROI, TC1, H, ROI), lambda b, t: (b, 0, t, 0, 0)),
            pl.BlockSpec((1, ROI, TC1, H, F), lambda b, t: (b, 0, t, 0, 0)),
            pl.BlockSpec((H, F, F), lambda b, t: (0, 0, 0)),
            pl.BlockSpec((H, 1, F), lambda b, t: (0, 0, 0)),
            pl.BlockSpec((H, 1, 1), lambda b, t: (0, 0, 0)),
        ],
        out_specs=[
            pl.BlockSpec((H, 1, TC1, ROI, F), lambda b, t: (0, b, t, 0, 0)),
            pl.BlockSpec((H, 8, F), lambda b, t: (0, 0, 0)),
        ],
        out_shape=[
            jax.ShapeDtypeStruct((H, B, T, ROI, F), jnp.bfloat16),
            jax.ShapeDtypeStruct((H, 8, F), jnp.float32),
        ],
        scratch_shapes=[pltpu.VMEM((H, 8, F), jnp.float32)],
    )(a, x, gnn_W1, gnn_b1.reshape(H, 1, F), gnn_eps.reshape(H, 1, 1))

    scale1, shift1 = _bn_affine(st1, gnn_g1, gnn_be1)
    z1p = jnp.transpose(z1, (0, 3, 2, 1, 4)).reshape(H, ROI, BT, F)

    z2r, st2 = pl.pallas_call(
        _k2_body,
        grid=(H, ROI // RT),
        in_specs=[
            pl.BlockSpec((1, RT, BT, F), lambda h, r: (h, r, 0, 0)),
            pl.BlockSpec((1, 1, F), lambda h, r: (h, 0, 0)),
            pl.BlockSpec((1, 1, F), lambda h, r: (h, 0, 0)),
            pl.BlockSpec((1, F, F), lambda h, r: (h, 0, 0)),
            pl.BlockSpec((1, 1, F), lambda h, r: (h, 0, 0)),
        ],
        out_specs=[
            pl.BlockSpec((1, RT, BT, F), lambda h, r: (h, r, 0, 0)),
            pl.BlockSpec((1, 8, F), lambda h, r: (h, 0, 0)),
        ],
        out_shape=[
            jax.ShapeDtypeStruct((H, ROI, BT, F), jnp.bfloat16),
            jax.ShapeDtypeStruct((H, 8, F), jnp.float32),
        ],
        scratch_shapes=[pltpu.VMEM((8, F), jnp.float32)],
    )(z1p, scale1, shift1, gnn_W2, gnn_b2.reshape(H, 1, F))

    scale2, shift2 = _bn_affine(st2, gnn_g2, gnn_be2)

    gi = pl.pallas_call(
        _k3_body,
        grid=(R // RT,),
        in_specs=[
            pl.BlockSpec((H, RT // H, BT, F), lambda c: (0, c, 0, 0)),
            pl.BlockSpec((H, 1, F), lambda c: (0, 0, 0)),
            pl.BlockSpec((H, 1, F), lambda c: (0, 0, 0)),
            pl.BlockSpec((RT, G, F), lambda c: (c, 0, 0)),
            pl.BlockSpec((RT, 1, G), lambda c: (c, 0, 0)),
        ],
        out_specs=pl.BlockSpec((T, RT, B, G), lambda c: (0, c, 0, 0)),
        out_shape=jax.ShapeDtypeStruct((T, R, B, G), jnp.bfloat16),
    )(z2r, scale2, shift2, gru_Wih, gru_bih.reshape(R, 1, G))

    gif = gi
    whh_h = jnp.transpose(gru_Whh.astype(jnp.bfloat16), (0, 2, 1))
    bhh_h = gru_bhh.reshape(R, 1, G)

    y = pl.pallas_call(
        _k4_body,
        grid=(NRC, T // TC4),
        in_specs=[
            pl.BlockSpec((TC4, RC, B, G), lambda c, t: (t, c, 0, 0)),
            pl.BlockSpec((RC, HID, G), lambda c, t: (c, 0, 0)),
            pl.BlockSpec((RC, 1, G), lambda c, t: (c, 0, 0)),
        ],
        out_specs=pl.BlockSpec((TC4, RC, B, HID), lambda c, t: (t, c, 0, 0)),
        out_shape=jax.ShapeDtypeStruct((T, R, B, HID), jnp.float32),
        scratch_shapes=[pltpu.VMEM((RC, B, HID), jnp.float32)],
    )(gif, whh_h, bhh_h)

    return jnp.transpose(y.reshape(T, ROI, H, B, HID), (3, 1, 0, 2, 4))
